# Initial kernel scaffold; baseline (speedup 1.0000x reference)
import jax
import jax.numpy as jnp
from jax.experimental import pallas as pl


def kernel(x_i, x_j, edge_index_inner_i, edge_index_inner_j, edge_index_outer_ij, edge_index_outer_ji, batch_i, batch_j, Wsrc, Wdst, att_src, att_dst, conv_bias, lin_W, lin_b):
    raise NotImplementedError("write your pallas kernel here")


# speedup vs baseline: 37.0226x; 37.0226x over previous
import functools

import jax
import jax.numpy as jnp
from jax import lax
from jax.experimental import pallas as pl
from jax.experimental.pallas import tpu as pltpu
from jax.experimental.pallas import tpu_sc as plsc

N = 10000
E = 320000
D = 128
BATCH = 16

NTILE = 16
EPT = 164 * 128
NCHUNK = 164
CK = 128
ETOT = 16 * EPT
ROWS = 10240
RPT = 640


def _proj_body(x_ref, ws_ref, wd_ref, att_ref, hs_ref, a_ref):
    x = x_ref[...]
    hs = jnp.dot(x, ws_ref[...], preferred_element_type=jnp.float32)
    hd = jnp.dot(x, wd_ref[...], preferred_element_type=jnp.float32)
    hs_ref[...] = hs
    att = att_ref[...]
    a0 = jnp.dot(hs[:, :128], att[0:1, :].T, preferred_element_type=jnp.float32)
    a1 = jnp.dot(hs[:, 128:], att[1:2, :].T, preferred_element_type=jnp.float32)
    a2 = jnp.dot(hd[:, :128], att[2:3, :].T, preferred_element_type=jnp.float32)
    a3 = jnp.dot(hd[:, 128:], att[3:4, :].T, preferred_element_type=jnp.float32)
    a_ref[...] = jnp.concatenate([a0, a1, a2, a3], axis=1)


def _project(x, ws_pair, wd_pair, att4):
    bn = 2000
    grid = N // bn
    return pl.pallas_call(
        _proj_body,
        grid=(grid,),
        in_specs=[
            pl.BlockSpec((bn, D), lambda i: (i, 0)),
            pl.BlockSpec((D, 2 * D), lambda i: (0, 0)),
            pl.BlockSpec((D, 2 * D), lambda i: (0, 0)),
            pl.BlockSpec((4, D), lambda i: (0, 0)),
        ],
        out_specs=[
            pl.BlockSpec((bn, 2 * D), lambda i: (i, 0)),
            pl.BlockSpec((bn, 4), lambda i: (i, 0)),
        ],
        out_shape=[
            jax.ShapeDtypeStruct((N, 2 * D), jnp.float32),
            jax.ShapeDtypeStruct((N, 4), jnp.float32),
        ],
    )(x, ws_pair, wd_pair, att4)


def _sc_pair_body(hs_hbm, as_hbm, ad_hbm, src_hbm, dst_hbm,
                  zrows_hbm, zvec_hbm, out_ref, den_ref,
                  asv, adv, srcb, dstb, rows, exbuf, acc, den_sh,
                  semes, semed, semr):
    c = lax.axis_index("c")
    s = lax.axis_index("s")
    w = c * NTILE + s
    pltpu.sync_copy(as_hbm.at[pl.ds(c * N, N)], asv)
    pltpu.sync_copy(ad_hbm.at[pl.ds(c * N, N)], adv)
    r0 = s * RPT
    pltpu.sync_copy(zrows_hbm, acc.at[pl.ds(r0, RPT)])
    pltpu.sync_copy(zvec_hbm, den_sh.at[pl.ds(r0, RPT)])
    plsc.subcore_barrier()

    def edge_start(g, b):
        row = w * NCHUNK + g
        pltpu.async_copy(src_hbm.at[row], srcb.at[b], semes)
        pltpu.async_copy(dst_hbm.at[row], dstb.at[b], semed)

    def edge_wait(b):
        pltpu.make_async_copy(src_hbm.at[0], srcb.at[b], semes).wait()
        pltpu.make_async_copy(dst_hbm.at[0], dstb.at[b], semed).wait()

    def compute_ex(b):
        def one(j, _):
            sv = srcb[b, pl.ds(j * 16, 16)]
            dv = dstb[b, pl.ds(j * 16, 16)]
            a = plsc.load_gather(asv, [sv]) + plsc.load_gather(adv, [dv])
            alpha = jnp.where(a >= 0.0, a, a * jnp.float32(0.2))
            exbuf[pl.ds(j * 16, 16)] = jnp.exp(alpha)
            return 0
        lax.fori_loop(0, CK // 16, one, 0, unroll=2)

    def scale_rows():
        def one(j, _):
            ex16 = exbuf[pl.ds(j * 16, 16)]
            for t in range(16):
                ei = j * 16 + t
                sc = ex16[t]
                for v in range(D // 16):
                    sl = pl.ds(v * 16, 16)
                    rows[ei, sl] = rows[ei, sl] * sc
            return 0
        lax.fori_loop(0, CK // 16, one, 0)

    def scatter(b):
        pltpu.sync_copy(rows, acc.at[dstb.at[b]], add=True)
        pltpu.sync_copy(exbuf, den_sh.at[dstb.at[b]], add=True)

    edge_start(0, 0)

    def step(g, b):
        edge_wait(b)
        cr = pltpu.async_copy(hs_hbm.at[srcb.at[b]], rows, semr)

        @pl.when(g + 1 < NCHUNK)
        def _():
            edge_start(g + 1, 1 - b)
        compute_ex(b)
        cr.wait()
        scale_rows()
        scatter(b)

    def step2(i, cc):
        step(i * 2, 0)
        step(i * 2 + 1, 1)
        return cc

    lax.fori_loop(0, NCHUNK // 2, step2, 0)

    plsc.subcore_barrier()
    o0 = c * ROWS + r0
    pltpu.sync_copy(acc.at[pl.ds(r0, RPT)], out_ref.at[pl.ds(o0, RPT)])
    pltpu.sync_copy(den_sh.at[pl.ds(r0, RPT)], den_ref.at[pl.ds(o0, RPT)])


def _sc_pair(hsA, asA, adA, eA, hsB, asB, adB, eB, zrows, zvec):
    hs_cat = jnp.concatenate([hsA, hsB], axis=0)
    as_cat = jnp.concatenate([asA, asB])
    ad_cat = jnp.concatenate([adA, adB])
    src_cat = jnp.concatenate([eA[0], eB[0] + N], axis=0)
    dst_cat = jnp.concatenate([eA[1], eB[1]], axis=0)
    mesh = plsc.VectorSubcoreMesh(core_axis_name="c", subcore_axis_name="s")
    fn = pl.kernel(
        _sc_pair_body,
        mesh=mesh,
        compiler_params=pltpu.CompilerParams(needs_layout_passes=False,
                                             use_tc_tiling_on_sc=False),
        out_type=[
            jax.ShapeDtypeStruct((2 * ROWS, D), jnp.float32),
            jax.ShapeDtypeStruct((2 * ROWS,), jnp.float32),
        ],
        scratch_types=[
            pltpu.VMEM((N,), jnp.float32),
            pltpu.VMEM((N,), jnp.float32),
            pltpu.VMEM((2, CK), jnp.int32),
            pltpu.VMEM((2, CK), jnp.int32),
            pltpu.VMEM((CK, D), jnp.float32),
            pltpu.VMEM((CK,), jnp.float32),
            pltpu.VMEM_SHARED((ROWS, D), jnp.float32),
            pltpu.VMEM_SHARED((ROWS,), jnp.float32),
            pltpu.SemaphoreType.DMA,
            pltpu.SemaphoreType.DMA,
            pltpu.SemaphoreType.DMA,
        ],
    )
    out, den = fn(hs_cat, as_cat, ad_cat, src_cat, dst_cat, zrows, zvec)
    out = out.reshape(2, ROWS, D)
    den = den.reshape(2, ROWS)
    return out[:, :N], den[:, :N]


def _epi_body(o_ref, d_ref, b_ref, h_ref):
    o = o_ref[...]
    den = d_ref[...]
    b = b_ref[...]
    eps = jnp.float32(1e-16)
    t0 = o[0] / (den[0] + eps) + b[0][None, :]
    t1 = o[1] / (den[1] + eps) + b[1][None, :]
    h_ref[...] = jnp.tanh(t0 + t1)


def _epilogue(out2, den2, bias2):
    bn = 2000
    grid = N // bn
    return pl.pallas_call(
        _epi_body,
        grid=(grid,),
        in_specs=[
            pl.BlockSpec((2, bn, D), lambda i: (0, i, 0)),
            pl.BlockSpec((2, bn, 1), lambda i: (0, i, 0)),
            pl.BlockSpec((2, D), lambda i: (0, 0)),
        ],
        out_specs=pl.BlockSpec((bn, D), lambda i: (i, 0)),
        out_shape=jax.ShapeDtypeStruct((N, D), jnp.float32),
    )(out2, den2.reshape(2, N, 1), bias2)


def _pool_body(hi_ref, hj_ref, bi_ref, bj_ref, w_ref, b_ref, o_ref,
               pi_ref, pj_ref, ci_ref, cj_ref):
    i = pl.program_id(0)

    @pl.when(i == 0)
    def _():
        pi_ref[...] = jnp.zeros_like(pi_ref)
        pj_ref[...] = jnp.zeros_like(pj_ref)
        ci_ref[...] = jnp.zeros_like(ci_ref)
        cj_ref[...] = jnp.zeros_like(cj_ref)

    iot = lax.broadcasted_iota(jnp.int32, (1, BATCH), 1)
    ones = jnp.ones((hi_ref.shape[0], D), jnp.float32)
    ohi = (bi_ref[...] == iot).astype(jnp.float32)
    ohj = (bj_ref[...] == iot).astype(jnp.float32)
    dn = (((0,), (0,)), ((), ()))
    pi_ref[...] += lax.dot_general(ohi, hi_ref[...], dn,
                                   preferred_element_type=jnp.float32)
    pj_ref[...] += lax.dot_general(ohj, hj_ref[...], dn,
                                   preferred_element_type=jnp.float32)
    ci_ref[...] += lax.dot_general(ohi, ones, dn,
                                   preferred_element_type=jnp.float32)
    cj_ref[...] += lax.dot_general(ohj, ones, dn,
                                   preferred_element_type=jnp.float32)

    @pl.when(i == pl.num_programs(0) - 1)
    def _():
        one = jnp.float32(1.0)
        pi = pi_ref[...] / jnp.maximum(ci_ref[...], one)
        pj = pj_ref[...] / jnp.maximum(cj_ref[...], one)
        x = (pi + pj) * jnp.float32(0.5)
        y = jnp.dot(x, w_ref[...], preferred_element_type=jnp.float32)
        o_ref[...] = jax.nn.sigmoid(y + b_ref[...][None, :])


def _pool_head(hi, hj, bi, bj, lin_W, lin_b):
    bn = 2000
    grid = N // bn
    return pl.pallas_call(
        _pool_body,
        grid=(grid,),
        in_specs=[
            pl.BlockSpec((bn, D), lambda i: (i, 0)),
            pl.BlockSpec((bn, D), lambda i: (i, 0)),
            pl.BlockSpec((bn, 1), lambda i: (i, 0)),
            pl.BlockSpec((bn, 1), lambda i: (i, 0)),
            pl.BlockSpec((D, 1), lambda i: (0, 0)),
            pl.BlockSpec((1,), lambda i: (0,)),
        ],
        out_specs=pl.BlockSpec((BATCH, 1), lambda i: (0, 0)),
        out_shape=jax.ShapeDtypeStruct((BATCH, 1), jnp.float32),
        scratch_shapes=[
            pltpu.VMEM((BATCH, D), jnp.float32),
            pltpu.VMEM((BATCH, D), jnp.float32),
            pltpu.VMEM((BATCH, D), jnp.float32),
            pltpu.VMEM((BATCH, D), jnp.float32),
        ],
    )(hi, hj, bi.reshape(N, 1), bj.reshape(N, 1), lin_W, lin_b)


def _prep_edges(ei):
    src, dst = ei[0], ei[1]
    npad = ETOT - E - N
    trash = N + (jnp.arange(E, dtype=jnp.int32) % 240)
    dst_eff = jnp.where(src != dst, dst, trash)
    loop = jnp.arange(N, dtype=jnp.int32)
    pad_src = (jnp.arange(npad, dtype=jnp.int32) * 97) % N
    pad_dst = N + (jnp.arange(npad, dtype=jnp.int32) % 240)
    src_full = jnp.concatenate([src, loop, pad_src]).reshape(NTILE * NCHUNK, CK)
    dst_full = jnp.concatenate([dst_eff, loop, pad_dst]).reshape(NTILE * NCHUNK, CK)
    return src_full.astype(jnp.int32), dst_full.astype(jnp.int32)


def kernel(x_i, x_j, edge_index_inner_i, edge_index_inner_j,
           edge_index_outer_ij, edge_index_outer_ji, batch_i, batch_j,
           Wsrc, Wdst, att_src, att_dst, conv_bias, lin_W, lin_b):
    e = [_prep_edges(edge_index_inner_i), _prep_edges(edge_index_inner_j),
         _prep_edges(edge_index_outer_ij), _prep_edges(edge_index_outer_ji)]
    zrows = jnp.zeros((RPT, D), jnp.float32)
    zvec = jnp.zeros((RPT,), jnp.float32)

    hi, hj = x_i, x_j
    for l in range(2):
        ws_i = jnp.concatenate([Wsrc[l, 0], Wsrc[l, 2]], axis=1)
        wd_i = jnp.concatenate([Wdst[l, 0], Wdst[l, 3]], axis=1)
        att_i = jnp.stack([att_src[l, 0], att_src[l, 2],
                           att_dst[l, 0], att_dst[l, 3]])
        hs_i, a_i = _project(hi, ws_i, wd_i, att_i)

        ws_j = jnp.concatenate([Wsrc[l, 1], Wsrc[l, 3]], axis=1)
        wd_j = jnp.concatenate([Wdst[l, 1], Wdst[l, 2]], axis=1)
        att_j = jnp.stack([att_src[l, 1], att_src[l, 3],
                           att_dst[l, 1], att_dst[l, 2]])
        hs_j, a_j = _project(hj, ws_j, wd_j, att_j)

        oi, di = _sc_pair(hs_i[:, :D], a_i[:, 0], a_i[:, 2], e[0],
                          hs_j[:, D:], a_j[:, 1], a_i[:, 3], e[3],
                          zrows, zvec)
        oj, dj = _sc_pair(hs_j[:, :D], a_j[:, 0], a_j[:, 2], e[1],
                          hs_i[:, D:], a_i[:, 1], a_j[:, 3], e[2],
                          zrows, zvec)

        bi2 = jnp.stack([conv_bias[l, 0], conv_bias[l, 3]])
        bj2 = jnp.stack([conv_bias[l, 1], conv_bias[l, 2]])
        hi = _epilogue(oi, di, bi2)
        hj = _epilogue(oj, dj, bj2)

    return _pool_head(hi, hj, batch_i.astype(jnp.int32),
                      batch_j.astype(jnp.int32), lin_W, lin_b)


# speedup vs baseline: 48.0531x; 1.2979x over previous
import functools

import jax
import jax.numpy as jnp
from jax import lax
from jax.experimental import pallas as pl
from jax.experimental.pallas import tpu as pltpu
from jax.experimental.pallas import tpu_sc as plsc

N = 10000
E = 320000
D = 128
BATCH = 16

NTILE = 16
EPT = 164 * 128
NCHUNK = 164
CK = 128
ETOT = 16 * EPT
ROWS = 10240
RPT = 640


def _proj_body(x_ref, ws_ref, wd_ref, att_ref, hs_ref, a_ref):
    x = x_ref[...]
    hs = jnp.dot(x, ws_ref[...], preferred_element_type=jnp.float32)
    hd = jnp.dot(x, wd_ref[...], preferred_element_type=jnp.float32)
    hs_ref[...] = hs
    att = att_ref[...]
    a0 = jnp.dot(hs[:, :128], att[0:1, :].T, preferred_element_type=jnp.float32)
    a1 = jnp.dot(hs[:, 128:], att[1:2, :].T, preferred_element_type=jnp.float32)
    a2 = jnp.dot(hd[:, :128], att[2:3, :].T, preferred_element_type=jnp.float32)
    a3 = jnp.dot(hd[:, 128:], att[3:4, :].T, preferred_element_type=jnp.float32)
    a_ref[...] = jnp.concatenate([a0, a1, a2, a3], axis=1)


def _project(x, ws_pair, wd_pair, att4):
    bn = 2000
    grid = N // bn
    return pl.pallas_call(
        _proj_body,
        grid=(grid,),
        in_specs=[
            pl.BlockSpec((bn, D), lambda i: (i, 0)),
            pl.BlockSpec((D, 2 * D), lambda i: (0, 0)),
            pl.BlockSpec((D, 2 * D), lambda i: (0, 0)),
            pl.BlockSpec((4, D), lambda i: (0, 0)),
        ],
        out_specs=[
            pl.BlockSpec((bn, 2 * D), lambda i: (i, 0)),
            pl.BlockSpec((bn, 4), lambda i: (i, 0)),
        ],
        out_shape=[
            jax.ShapeDtypeStruct((N, 2 * D), jnp.float32),
            jax.ShapeDtypeStruct((N, 4), jnp.float32),
        ],
    )(x, ws_pair, wd_pair, att4)


def _sc_pair_body(hs_hbm, as_hbm, ad_hbm, src_hbm, dst_hbm,
                  zrows_hbm, zvec_hbm, out_ref, den_ref,
                  srcb, dstb, rows, exbuf, asb, adb,
                  acc, den_sh, as_sh, ad_sh,
                  semes, semed, semr, sema, semd):
    c = lax.axis_index("c")
    s = lax.axis_index("s")
    w = c * NTILE + s
    @pl.when(s == 0)
    def _():
        pltpu.sync_copy(as_hbm.at[pl.ds(c * N, N)], as_sh)
        pltpu.sync_copy(ad_hbm.at[pl.ds(c * N, N)], ad_sh)
    r0 = s * RPT
    pltpu.sync_copy(zrows_hbm, acc.at[pl.ds(r0, RPT)])
    pltpu.sync_copy(zvec_hbm, den_sh.at[pl.ds(r0, RPT)])
    plsc.subcore_barrier()

    def edge_start(g, b):
        row = w * NCHUNK + g
        pltpu.async_copy(src_hbm.at[row], srcb.at[b], semes)
        pltpu.async_copy(dst_hbm.at[row], dstb.at[b], semed)

    def edge_wait(b):
        pltpu.make_async_copy(src_hbm.at[0], srcb.at[b], semes).wait()
        pltpu.make_async_copy(dst_hbm.at[0], dstb.at[b], semed).wait()

    def gathers_start(b):
        pltpu.async_copy(hs_hbm.at[srcb.at[b]], rows.at[b], semr)
        pltpu.async_copy(as_sh.at[srcb.at[b]], asb.at[b], sema)
        pltpu.async_copy(ad_sh.at[dstb.at[b]], adb.at[b], semd)

    def gathers_wait(b):
        pltpu.make_async_copy(hs_hbm.at[srcb.at[b]], rows.at[b], semr).wait()
        pltpu.make_async_copy(as_sh.at[srcb.at[b]], asb.at[b], sema).wait()
        pltpu.make_async_copy(ad_sh.at[dstb.at[b]], adb.at[b], semd).wait()

    def compute_ex(b):
        def one(j, _):
            a = asb[b, pl.ds(j * 16, 16)] + adb[b, pl.ds(j * 16, 16)]
            alpha = jnp.where(a >= 0.0, a, a * jnp.float32(0.2))
            exbuf[pl.ds(j * 16, 16)] = jnp.exp(alpha)
            return 0
        lax.fori_loop(0, CK // 16, one, 0, unroll=2)

    def scale_rows(b):
        def one(j, _):
            ex16 = exbuf[pl.ds(j * 16, 16)]
            for t in range(16):
                ei = j * 16 + t
                sc = ex16[t]
                for v in range(D // 16):
                    sl = pl.ds(v * 16, 16)
                    rows[b, ei, sl] = rows[b, ei, sl] * sc
            return 0
        lax.fori_loop(0, CK // 16, one, 0)

    def scatter(b):
        pltpu.sync_copy(rows.at[b], acc.at[dstb.at[b]], add=True)
        pltpu.sync_copy(exbuf, den_sh.at[dstb.at[b]], add=True)

    edge_start(0, 0)
    edge_wait(0)
    gathers_start(0)
    edge_start(1, 1)

    def step(g, b):
        @pl.when(g + 1 < NCHUNK)
        def _():
            edge_wait(1 - b)
            gathers_start(1 - b)
        gathers_wait(b)
        compute_ex(b)
        scale_rows(b)
        scatter(b)

        @pl.when(g + 2 < NCHUNK)
        def _():
            edge_start(g + 2, b)

    def step2(i, cc):
        step(i * 2, 0)
        step(i * 2 + 1, 1)
        return cc

    lax.fori_loop(0, NCHUNK // 2, step2, 0)

    plsc.subcore_barrier()
    o0 = c * ROWS + r0
    pltpu.sync_copy(acc.at[pl.ds(r0, RPT)], out_ref.at[pl.ds(o0, RPT)])
    pltpu.sync_copy(den_sh.at[pl.ds(r0, RPT)], den_ref.at[pl.ds(o0, RPT)])


def _sc_pair(hsA, asA, adA, eA, hsB, asB, adB, eB, zrows, zvec):
    hs_cat = jnp.concatenate([hsA, hsB], axis=0)
    as_cat = jnp.concatenate([asA, asB])
    ad_cat = jnp.concatenate([adA, adB])
    src_cat = jnp.concatenate([eA[0], eB[0] + N], axis=0)
    dst_cat = jnp.concatenate([eA[1], eB[1]], axis=0)
    mesh = plsc.VectorSubcoreMesh(core_axis_name="c", subcore_axis_name="s")
    fn = pl.kernel(
        _sc_pair_body,
        mesh=mesh,
        compiler_params=pltpu.CompilerParams(needs_layout_passes=False,
                                             use_tc_tiling_on_sc=False),
        out_type=[
            jax.ShapeDtypeStruct((2 * ROWS, D), jnp.float32),
            jax.ShapeDtypeStruct((2 * ROWS,), jnp.float32),
        ],
        scratch_types=[
            pltpu.VMEM((2, CK), jnp.int32),
            pltpu.VMEM((2, CK), jnp.int32),
            pltpu.VMEM((2, CK, D), jnp.float32),
            pltpu.VMEM((CK,), jnp.float32),
            pltpu.VMEM((2, CK), jnp.float32),
            pltpu.VMEM((2, CK), jnp.float32),
            pltpu.VMEM_SHARED((ROWS, D), jnp.float32),
            pltpu.VMEM_SHARED((ROWS,), jnp.float32),
            pltpu.VMEM_SHARED((N,), jnp.float32),
            pltpu.VMEM_SHARED((N,), jnp.float32),
            pltpu.SemaphoreType.DMA,
            pltpu.SemaphoreType.DMA,
            pltpu.SemaphoreType.DMA,
            pltpu.SemaphoreType.DMA,
            pltpu.SemaphoreType.DMA,
        ],
    )
    out, den = fn(hs_cat, as_cat, ad_cat, src_cat, dst_cat, zrows, zvec)
    out = out.reshape(2, ROWS, D)
    den = den.reshape(2, ROWS)
    return out[:, :N], den[:, :N]


def _epi_body(o_ref, d_ref, b_ref, h_ref):
    o = o_ref[...]
    den = d_ref[...]
    b = b_ref[...]
    eps = jnp.float32(1e-16)
    t0 = o[0] / (den[0] + eps) + b[0][None, :]
    t1 = o[1] / (den[1] + eps) + b[1][None, :]
    h_ref[...] = jnp.tanh(t0 + t1)


def _epilogue(out2, den2, bias2):
    bn = 2000
    grid = N // bn
    return pl.pallas_call(
        _epi_body,
        grid=(grid,),
        in_specs=[
            pl.BlockSpec((2, bn, D), lambda i: (0, i, 0)),
            pl.BlockSpec((2, bn, 1), lambda i: (0, i, 0)),
            pl.BlockSpec((2, D), lambda i: (0, 0)),
        ],
        out_specs=pl.BlockSpec((bn, D), lambda i: (i, 0)),
        out_shape=jax.ShapeDtypeStruct((N, D), jnp.float32),
    )(out2, den2.reshape(2, N, 1), bias2)


def _pool_body(hi_ref, hj_ref, bi_ref, bj_ref, w_ref, b_ref, o_ref,
               pi_ref, pj_ref, ci_ref, cj_ref):
    i = pl.program_id(0)

    @pl.when(i == 0)
    def _():
        pi_ref[...] = jnp.zeros_like(pi_ref)
        pj_ref[...] = jnp.zeros_like(pj_ref)
        ci_ref[...] = jnp.zeros_like(ci_ref)
        cj_ref[...] = jnp.zeros_like(cj_ref)

    iot = lax.broadcasted_iota(jnp.int32, (1, BATCH), 1)
    ones = jnp.ones((hi_ref.shape[0], D), jnp.float32)
    ohi = (bi_ref[...] == iot).astype(jnp.float32)
    ohj = (bj_ref[...] == iot).astype(jnp.float32)
    dn = (((0,), (0,)), ((), ()))
    pi_ref[...] += lax.dot_general(ohi, hi_ref[...], dn,
                                   preferred_element_type=jnp.float32)
    pj_ref[...] += lax.dot_general(ohj, hj_ref[...], dn,
                                   preferred_element_type=jnp.float32)
    ci_ref[...] += lax.dot_general(ohi, ones, dn,
                                   preferred_element_type=jnp.float32)
    cj_ref[...] += lax.dot_general(ohj, ones, dn,
                                   preferred_element_type=jnp.float32)

    @pl.when(i == pl.num_programs(0) - 1)
    def _():
        one = jnp.float32(1.0)
        pi = pi_ref[...] / jnp.maximum(ci_ref[...], one)
        pj = pj_ref[...] / jnp.maximum(cj_ref[...], one)
        x = (pi + pj) * jnp.float32(0.5)
        y = jnp.dot(x, w_ref[...], preferred_element_type=jnp.float32)
        o_ref[...] = jax.nn.sigmoid(y + b_ref[...][None, :])


def _pool_head(hi, hj, bi, bj, lin_W, lin_b):
    bn = 2000
    grid = N // bn
    return pl.pallas_call(
        _pool_body,
        grid=(grid,),
        in_specs=[
            pl.BlockSpec((bn, D), lambda i: (i, 0)),
            pl.BlockSpec((bn, D), lambda i: (i, 0)),
            pl.BlockSpec((bn, 1), lambda i: (i, 0)),
            pl.BlockSpec((bn, 1), lambda i: (i, 0)),
            pl.BlockSpec((D, 1), lambda i: (0, 0)),
            pl.BlockSpec((1,), lambda i: (0,)),
        ],
        out_specs=pl.BlockSpec((BATCH, 1), lambda i: (0, 0)),
        out_shape=jax.ShapeDtypeStruct((BATCH, 1), jnp.float32),
        scratch_shapes=[
            pltpu.VMEM((BATCH, D), jnp.float32),
            pltpu.VMEM((BATCH, D), jnp.float32),
            pltpu.VMEM((BATCH, D), jnp.float32),
            pltpu.VMEM((BATCH, D), jnp.float32),
        ],
    )(hi, hj, bi.reshape(N, 1), bj.reshape(N, 1), lin_W, lin_b)


def _prep_edges(ei):
    src, dst = ei[0], ei[1]
    npad = ETOT - E - N
    trash = N + (jnp.arange(E, dtype=jnp.int32) % 240)
    dst_eff = jnp.where(src != dst, dst, trash)
    loop = jnp.arange(N, dtype=jnp.int32)
    pad_src = (jnp.arange(npad, dtype=jnp.int32) * 97) % N
    pad_dst = N + (jnp.arange(npad, dtype=jnp.int32) % 240)
    src_full = jnp.concatenate([src, loop, pad_src]).reshape(NTILE * NCHUNK, CK)
    dst_full = jnp.concatenate([dst_eff, loop, pad_dst]).reshape(NTILE * NCHUNK, CK)
    return src_full.astype(jnp.int32), dst_full.astype(jnp.int32)


def kernel(x_i, x_j, edge_index_inner_i, edge_index_inner_j,
           edge_index_outer_ij, edge_index_outer_ji, batch_i, batch_j,
           Wsrc, Wdst, att_src, att_dst, conv_bias, lin_W, lin_b):
    e = [_prep_edges(edge_index_inner_i), _prep_edges(edge_index_inner_j),
         _prep_edges(edge_index_outer_ij), _prep_edges(edge_index_outer_ji)]
    zrows = jnp.zeros((RPT, D), jnp.float32)
    zvec = jnp.zeros((RPT,), jnp.float32)

    hi, hj = x_i, x_j
    for l in range(2):
        ws_i = jnp.concatenate([Wsrc[l, 0], Wsrc[l, 2]], axis=1)
        wd_i = jnp.concatenate([Wdst[l, 0], Wdst[l, 3]], axis=1)
        att_i = jnp.stack([att_src[l, 0], att_src[l, 2],
                           att_dst[l, 0], att_dst[l, 3]])
        hs_i, a_i = _project(hi, ws_i, wd_i, att_i)

        ws_j = jnp.concatenate([Wsrc[l, 1], Wsrc[l, 3]], axis=1)
        wd_j = jnp.concatenate([Wdst[l, 1], Wdst[l, 2]], axis=1)
        att_j = jnp.stack([att_src[l, 1], att_src[l, 3],
                           att_dst[l, 1], att_dst[l, 2]])
        hs_j, a_j = _project(hj, ws_j, wd_j, att_j)

        oi, di = _sc_pair(hs_i[:, :D], a_i[:, 0], a_i[:, 2], e[0],
                          hs_j[:, D:], a_j[:, 1], a_i[:, 3], e[3],
                          zrows, zvec)
        oj, dj = _sc_pair(hs_j[:, :D], a_j[:, 0], a_j[:, 2], e[1],
                          hs_i[:, D:], a_i[:, 1], a_j[:, 3], e[2],
                          zrows, zvec)

        bi2 = jnp.stack([conv_bias[l, 0], conv_bias[l, 3]])
        bj2 = jnp.stack([conv_bias[l, 1], conv_bias[l, 2]])
        hi = _epilogue(oi, di, bi2)
        hj = _epilogue(oj, dj, bj2)

    return _pool_head(hi, hj, batch_i.astype(jnp.int32),
                      batch_j.astype(jnp.int32), lin_W, lin_b)


# speedup vs baseline: 58.7037x; 1.2216x over previous
import functools

import jax
import jax.numpy as jnp
from jax import lax
from jax.experimental import pallas as pl
from jax.experimental.pallas import tpu as pltpu
from jax.experimental.pallas import tpu_sc as plsc

N = 10000
E = 320000
D = 128
BATCH = 16

NTILE = 16
EPT = 164 * 128
NCHUNK = 164
CK = 128
ETOT = 16 * EPT
ROWS = 10240
RPT = 640


def _proj_body(x_ref, ws_ref, wd_ref, att_ref, hs_ref, a_ref):
    x = x_ref[...]
    hs = jnp.dot(x, ws_ref[...], preferred_element_type=jnp.float32)
    hd = jnp.dot(x, wd_ref[...], preferred_element_type=jnp.float32)
    hs_ref[...] = hs
    att = att_ref[...]
    a0 = jnp.dot(hs[:, :128], att[0:1, :].T, preferred_element_type=jnp.float32)
    a1 = jnp.dot(hs[:, 128:], att[1:2, :].T, preferred_element_type=jnp.float32)
    a2 = jnp.dot(hd[:, :128], att[2:3, :].T, preferred_element_type=jnp.float32)
    a3 = jnp.dot(hd[:, 128:], att[3:4, :].T, preferred_element_type=jnp.float32)
    a_ref[...] = jnp.concatenate([a0, a1, a2, a3], axis=1)


def _project(x, ws_pair, wd_pair, att4):
    bn = 2000
    grid = N // bn
    return pl.pallas_call(
        _proj_body,
        grid=(grid,),
        in_specs=[
            pl.BlockSpec((bn, D), lambda i: (i, 0)),
            pl.BlockSpec((D, 2 * D), lambda i: (0, 0)),
            pl.BlockSpec((D, 2 * D), lambda i: (0, 0)),
            pl.BlockSpec((4, D), lambda i: (0, 0)),
        ],
        out_specs=[
            pl.BlockSpec((bn, 2 * D), lambda i: (i, 0)),
            pl.BlockSpec((bn, 4), lambda i: (i, 0)),
        ],
        out_shape=[
            jax.ShapeDtypeStruct((N, 2 * D), jnp.float32),
            jax.ShapeDtypeStruct((N, 4), jnp.float32),
        ],
    )(x, ws_pair, wd_pair, att4)


def _sc_pair_body(hs_hbm, as_hbm, ad_hbm, src_hbm, dst_hbm,
                  zrows_hbm, zvec_hbm, out_ref, den_ref,
                  srcb, dstb, rows, exbuf, asb, adb,
                  acc, den_sh, as_sh, ad_sh,
                  semes, semed, semr, sema, semd, semsr, semsd):
    c = lax.axis_index("c")
    s = lax.axis_index("s")
    w = c * NTILE + s
    @pl.when(s == 0)
    def _():
        pltpu.sync_copy(as_hbm.at[pl.ds(c * N, N)], as_sh)
        pltpu.sync_copy(ad_hbm.at[pl.ds(c * N, N)], ad_sh)
    r0 = s * RPT
    pltpu.sync_copy(zrows_hbm, acc.at[pl.ds(r0, RPT)])
    pltpu.sync_copy(zvec_hbm, den_sh.at[pl.ds(r0, RPT)])
    plsc.subcore_barrier()

    def edge_start(g, e):
        row = w * NCHUNK + g
        pltpu.async_copy(src_hbm.at[row], srcb.at[e], semes)
        pltpu.async_copy(dst_hbm.at[row], dstb.at[e], semed)

    def edge_wait(e):
        pltpu.make_async_copy(src_hbm.at[0], srcb.at[e], semes).wait()
        pltpu.make_async_copy(dst_hbm.at[0], dstb.at[e], semed).wait()

    def gathers_start(b, e):
        pltpu.async_copy(hs_hbm.at[srcb.at[e]], rows.at[b], semr)
        pltpu.async_copy(as_sh.at[srcb.at[e]], asb.at[b], sema)
        pltpu.async_copy(ad_sh.at[dstb.at[e]], adb.at[b], semd)

    def gathers_wait(b, e):
        pltpu.make_async_copy(hs_hbm.at[srcb.at[e]], rows.at[b], semr).wait()
        pltpu.make_async_copy(as_sh.at[srcb.at[e]], asb.at[b], sema).wait()
        pltpu.make_async_copy(ad_sh.at[dstb.at[e]], adb.at[b], semd).wait()

    def compute_ex(b):
        def one(j, _):
            a = asb[b, pl.ds(j * 16, 16)] + adb[b, pl.ds(j * 16, 16)]
            alpha = jnp.where(a >= 0.0, a, a * jnp.float32(0.2))
            exbuf[b, pl.ds(j * 16, 16)] = jnp.exp(alpha)
            return 0
        lax.fori_loop(0, CK // 16, one, 0, unroll=2)

    def scale_rows(b):
        def one(j, _):
            ex16 = exbuf[b, pl.ds(j * 16, 16)]
            for t in range(16):
                ei = j * 16 + t
                sc = ex16[t]
                for v in range(D // 16):
                    sl = pl.ds(v * 16, 16)
                    rows[b, ei, sl] = rows[b, ei, sl] * sc
            return 0
        lax.fori_loop(0, CK // 16, one, 0)

    def scatter_start(b, e):
        pltpu.async_copy(rows.at[b], acc.at[dstb.at[e]], semsr, add=True)
        pltpu.async_copy(exbuf.at[b], den_sh.at[dstb.at[e]], semsd, add=True)

    def scatter_wait(b, e):
        pltpu.make_async_copy(rows.at[b], acc.at[dstb.at[e]], semsr).wait()
        pltpu.make_async_copy(exbuf.at[b], den_sh.at[dstb.at[e]], semsd).wait()

    edge_start(0, 0)
    edge_wait(0)
    gathers_start(0, 0)
    edge_start(1, 1)
    edge_start(2, 2)

    def step(g, b, e):
        @pl.when(g >= 1)
        def _():
            scatter_wait(1 - b, (e + 3) % 4)

        @pl.when(g + 1 < NCHUNK)
        def _():
            edge_wait((e + 1) % 4)
            gathers_start(1 - b, (e + 1) % 4)
        gathers_wait(b, e)
        compute_ex(b)
        scale_rows(b)
        scatter_start(b, e)

        @pl.when(g + 3 < NCHUNK)
        def _():
            edge_start(g + 3, (e + 3) % 4)

    def step4(i, cc):
        for k in range(4):
            step(i * 4 + k, k % 2, k)
        return cc

    lax.fori_loop(0, NCHUNK // 4, step4, 0)
    scatter_wait((NCHUNK - 1) % 2, (NCHUNK - 1) % 4)

    plsc.subcore_barrier()
    o0 = c * ROWS + r0
    pltpu.sync_copy(acc.at[pl.ds(r0, RPT)], out_ref.at[pl.ds(o0, RPT)])
    pltpu.sync_copy(den_sh.at[pl.ds(r0, RPT)], den_ref.at[pl.ds(o0, RPT)])


def _sc_pair(hsA, asA, adA, eA, hsB, asB, adB, eB, zrows, zvec):
    hs_cat = jnp.concatenate([hsA, hsB], axis=0)
    as_cat = jnp.concatenate([asA, asB])
    ad_cat = jnp.concatenate([adA, adB])
    src_cat = jnp.concatenate([eA[0], eB[0] + N], axis=0)
    dst_cat = jnp.concatenate([eA[1], eB[1]], axis=0)
    mesh = plsc.VectorSubcoreMesh(core_axis_name="c", subcore_axis_name="s")
    fn = pl.kernel(
        _sc_pair_body,
        mesh=mesh,
        compiler_params=pltpu.CompilerParams(needs_layout_passes=False,
                                             use_tc_tiling_on_sc=False),
        out_type=[
            jax.ShapeDtypeStruct((2 * ROWS, D), jnp.float32),
            jax.ShapeDtypeStruct((2 * ROWS,), jnp.float32),
        ],
        scratch_types=[
            pltpu.VMEM((4, CK), jnp.int32),
            pltpu.VMEM((4, CK), jnp.int32),
            pltpu.VMEM((2, CK, D), jnp.float32),
            pltpu.VMEM((2, CK), jnp.float32),
            pltpu.VMEM((2, CK), jnp.float32),
            pltpu.VMEM((2, CK), jnp.float32),
            pltpu.VMEM_SHARED((ROWS, D), jnp.float32),
            pltpu.VMEM_SHARED((ROWS,), jnp.float32),
            pltpu.VMEM_SHARED((N,), jnp.float32),
            pltpu.VMEM_SHARED((N,), jnp.float32),
            pltpu.SemaphoreType.DMA,
            pltpu.SemaphoreType.DMA,
            pltpu.SemaphoreType.DMA,
            pltpu.SemaphoreType.DMA,
            pltpu.SemaphoreType.DMA,
            pltpu.SemaphoreType.DMA,
            pltpu.SemaphoreType.DMA,
        ],
    )
    out, den = fn(hs_cat, as_cat, ad_cat, src_cat, dst_cat, zrows, zvec)
    out = out.reshape(2, ROWS, D)
    den = den.reshape(2, ROWS)
    return out[:, :N], den[:, :N]


def _epi_body(o_ref, d_ref, b_ref, h_ref):
    o = o_ref[...]
    den = d_ref[...]
    b = b_ref[...]
    eps = jnp.float32(1e-16)
    t0 = o[0] / (den[0] + eps) + b[0][None, :]
    t1 = o[1] / (den[1] + eps) + b[1][None, :]
    h_ref[...] = jnp.tanh(t0 + t1)


def _epilogue(out2, den2, bias2):
    bn = 2000
    grid = N // bn
    return pl.pallas_call(
        _epi_body,
        grid=(grid,),
        in_specs=[
            pl.BlockSpec((2, bn, D), lambda i: (0, i, 0)),
            pl.BlockSpec((2, bn, 1), lambda i: (0, i, 0)),
            pl.BlockSpec((2, D), lambda i: (0, 0)),
        ],
        out_specs=pl.BlockSpec((bn, D), lambda i: (i, 0)),
        out_shape=jax.ShapeDtypeStruct((N, D), jnp.float32),
    )(out2, den2.reshape(2, N, 1), bias2)


def _pool_body(hi_ref, hj_ref, bi_ref, bj_ref, w_ref, b_ref, o_ref,
               pi_ref, pj_ref, ci_ref, cj_ref):
    i = pl.program_id(0)

    @pl.when(i == 0)
    def _():
        pi_ref[...] = jnp.zeros_like(pi_ref)
        pj_ref[...] = jnp.zeros_like(pj_ref)
        ci_ref[...] = jnp.zeros_like(ci_ref)
        cj_ref[...] = jnp.zeros_like(cj_ref)

    iot = lax.broadcasted_iota(jnp.int32, (1, BATCH), 1)
    ones = jnp.ones((hi_ref.shape[0], D), jnp.float32)
    ohi = (bi_ref[...] == iot).astype(jnp.float32)
    ohj = (bj_ref[...] == iot).astype(jnp.float32)
    dn = (((0,), (0,)), ((), ()))
    pi_ref[...] += lax.dot_general(ohi, hi_ref[...], dn,
                                   preferred_element_type=jnp.float32)
    pj_ref[...] += lax.dot_general(ohj, hj_ref[...], dn,
                                   preferred_element_type=jnp.float32)
    ci_ref[...] += lax.dot_general(ohi, ones, dn,
                                   preferred_element_type=jnp.float32)
    cj_ref[...] += lax.dot_general(ohj, ones, dn,
                                   preferred_element_type=jnp.float32)

    @pl.when(i == pl.num_programs(0) - 1)
    def _():
        one = jnp.float32(1.0)
        pi = pi_ref[...] / jnp.maximum(ci_ref[...], one)
        pj = pj_ref[...] / jnp.maximum(cj_ref[...], one)
        x = (pi + pj) * jnp.float32(0.5)
        y = jnp.dot(x, w_ref[...], preferred_element_type=jnp.float32)
        o_ref[...] = jax.nn.sigmoid(y + b_ref[...][None, :])


def _pool_head(hi, hj, bi, bj, lin_W, lin_b):
    bn = 2000
    grid = N // bn
    return pl.pallas_call(
        _pool_body,
        grid=(grid,),
        in_specs=[
            pl.BlockSpec((bn, D), lambda i: (i, 0)),
            pl.BlockSpec((bn, D), lambda i: (i, 0)),
            pl.BlockSpec((bn, 1), lambda i: (i, 0)),
            pl.BlockSpec((bn, 1), lambda i: (i, 0)),
            pl.BlockSpec((D, 1), lambda i: (0, 0)),
            pl.BlockSpec((1,), lambda i: (0,)),
        ],
        out_specs=pl.BlockSpec((BATCH, 1), lambda i: (0, 0)),
        out_shape=jax.ShapeDtypeStruct((BATCH, 1), jnp.float32),
        scratch_shapes=[
            pltpu.VMEM((BATCH, D), jnp.float32),
            pltpu.VMEM((BATCH, D), jnp.float32),
            pltpu.VMEM((BATCH, D), jnp.float32),
            pltpu.VMEM((BATCH, D), jnp.float32),
        ],
    )(hi, hj, bi.reshape(N, 1), bj.reshape(N, 1), lin_W, lin_b)


def _prep_edges(ei):
    src, dst = ei[0], ei[1]
    npad = ETOT - E - N
    trash = N + (jnp.arange(E, dtype=jnp.int32) % 240)
    dst_eff = jnp.where(src != dst, dst, trash)
    loop = jnp.arange(N, dtype=jnp.int32)
    pad_src = (jnp.arange(npad, dtype=jnp.int32) * 97) % N
    pad_dst = N + (jnp.arange(npad, dtype=jnp.int32) % 240)
    src_full = jnp.concatenate([src, loop, pad_src]).reshape(NTILE * NCHUNK, CK)
    dst_full = jnp.concatenate([dst_eff, loop, pad_dst]).reshape(NTILE * NCHUNK, CK)
    return src_full.astype(jnp.int32), dst_full.astype(jnp.int32)


def kernel(x_i, x_j, edge_index_inner_i, edge_index_inner_j,
           edge_index_outer_ij, edge_index_outer_ji, batch_i, batch_j,
           Wsrc, Wdst, att_src, att_dst, conv_bias, lin_W, lin_b):
    e = [_prep_edges(edge_index_inner_i), _prep_edges(edge_index_inner_j),
         _prep_edges(edge_index_outer_ij), _prep_edges(edge_index_outer_ji)]
    zrows = jnp.zeros((RPT, D), jnp.float32)
    zvec = jnp.zeros((RPT,), jnp.float32)

    hi, hj = x_i, x_j
    for l in range(2):
        ws_i = jnp.concatenate([Wsrc[l, 0], Wsrc[l, 2]], axis=1)
        wd_i = jnp.concatenate([Wdst[l, 0], Wdst[l, 3]], axis=1)
        att_i = jnp.stack([att_src[l, 0], att_src[l, 2],
                           att_dst[l, 0], att_dst[l, 3]])
        hs_i, a_i = _project(hi, ws_i, wd_i, att_i)

        ws_j = jnp.concatenate([Wsrc[l, 1], Wsrc[l, 3]], axis=1)
        wd_j = jnp.concatenate([Wdst[l, 1], Wdst[l, 2]], axis=1)
        att_j = jnp.stack([att_src[l, 1], att_src[l, 3],
                           att_dst[l, 1], att_dst[l, 2]])
        hs_j, a_j = _project(hj, ws_j, wd_j, att_j)

        oi, di = _sc_pair(hs_i[:, :D], a_i[:, 0], a_i[:, 2], e[0],
                          hs_j[:, D:], a_j[:, 1], a_i[:, 3], e[3],
                          zrows, zvec)
        oj, dj = _sc_pair(hs_j[:, :D], a_j[:, 0], a_j[:, 2], e[1],
                          hs_i[:, D:], a_i[:, 1], a_j[:, 3], e[2],
                          zrows, zvec)

        bi2 = jnp.stack([conv_bias[l, 0], conv_bias[l, 3]])
        bj2 = jnp.stack([conv_bias[l, 1], conv_bias[l, 2]])
        hi = _epilogue(oi, di, bi2)
        hj = _epilogue(oj, dj, bj2)

    return _pool_head(hi, hj, batch_i.astype(jnp.int32),
                      batch_j.astype(jnp.int32), lin_W, lin_b)
